# baseline (speedup 1.0000x reference)
import functools

import jax
import jax.numpy as jnp
from jax import lax
from jax.experimental import pallas as pl
from jax.experimental.pallas import tpu as pltpu
from jax.experimental.pallas import tpu_sc as plsc

_INFO = plsc.get_sparse_core_info()
_NC = _INFO.num_cores
_NS = _INFO.num_subcores
_NW = _NC * _NS

_C = 256


@jax.jit
def _sc_lookup(table, idx_t):
    S, T = idx_t.shape
    V, D = table.shape
    t_per_w = T // _NW
    n_chunks = S * (t_per_w // _C)
    mesh = plsc.VectorSubcoreMesh(core_axis_name="c", subcore_axis_name="s")

    @functools.partial(
        pl.kernel,
        mesh=mesh,
        compiler_params=pltpu.CompilerParams(
            use_tc_tiling_on_sc=False, needs_layout_passes=False),
        out_type=jax.ShapeDtypeStruct((S, D // 8, T // 128, 8, 128),
                                      jnp.float32),
        scratch_types=[
            pltpu.VMEM((2, 1, _C), jnp.int32),
            pltpu.VMEM((_C, D), jnp.float32),
            pltpu.VMEM((_C, D), jnp.float32),
            pltpu.VMEM((D // 8, _C // 128, 8, 128), jnp.float32),
            pltpu.VMEM((D // 8, _C // 128, 8, 128), jnp.float32),
            pltpu.SemaphoreType.DMA,
            pltpu.SemaphoreType.DMA,
            pltpu.SemaphoreType.DMA,
            pltpu.SemaphoreType.DMA,
            pltpu.SemaphoreType.DMA,
            pltpu.SemaphoreType.DMA,
        ],
    )
    def k(table_hbm, idx_hbm, out_hbm, idx_v, rows0, rows1, tb0, tb1,
          gs0, gs1, ws0, ws1, is0, is1):
        wid = lax.axis_index("s") * _NC + lax.axis_index("c")
        t0w = wid * t_per_w
        rows = (rows0, rows1)
        tbs = (tb0, tb1)
        gsem = (gs0, gs1)
        wsem = (ws0, ws1)
        isem = (is0, is1)
        cpb = t_per_w // _C
        lane = lax.iota(jnp.int32, 16)
        ones = jnp.ones((16,), jnp.int32)

        def idx_load(n, b):
            s = n // cpb
            toff = t0w + (n % cpb) * _C
            return pltpu.make_async_copy(
                idx_hbm.at[pl.ds(s, 1), pl.ds(toff, _C)], idx_v.at[b],
                isem[b])

        def gather(b):
            return pltpu.make_async_copy(
                table_hbm.at[idx_v.at[b, 0]], rows[b], gsem[b])

        def block_write(n, b):
            s = n // cpb
            tt0 = wid * (t_per_w // 128) + (n % cpb) * (_C // 128)
            return pltpu.make_async_copy(
                tbs[b], out_hbm.at[s, :, pl.ds(tt0, _C // 128), :, :],
                wsem[b])

        def transpose(b):
            rb = rows[b]
            tb = tbs[b]

            def body(m, carry):
                g = m >> 4
                tj = (m >> 3) & 1
                dd = m & 7
                col = ones * (8 * g + dd)
                rbase = lane + tj * 128
                for kk in range(8):
                    v = plsc.load_gather(rb, [rbase + kk * 16, col])
                    tb[g, tj, dd, pl.ds(kk * 16, 16)] = v
                return carry

            lax.fori_loop(0, (D // 8) * (_C // 128) * 8, body, 0)

        idx_load(0, 0).start()
        idx_load(1, 1).start()
        idx_load(0, 0).wait()
        gather(0).start()

        def chunk(n, b):
            gather(b).wait()

            @pl.when(n + 2 < n_chunks)
            def _():
                idx_load(n + 2, b).start()

            @pl.when(n + 1 < n_chunks)
            def _():
                idx_load(n + 1, 1 - b).wait()
                gather(1 - b).start()

            @pl.when(n >= 2)
            def _():
                block_write(n - 2, b).wait()
            transpose(b)
            block_write(n, b).start()

        def pair(j, carry):
            chunk(2 * j, 0)
            chunk(2 * j + 1, 1)
            return carry

        lax.fori_loop(0, n_chunks // 2, pair, 0)
        block_write(n_chunks - 2, 0).wait()
        block_write(n_chunks - 1, 1).wait()

    return k(table, idx_t)


def kernel(inputs, embedding_weights):
    B0, B1 = inputs.shape
    V, D = embedding_weights.shape
    idx_t = jnp.swapaxes(inputs, 0, 1).astype(jnp.int32)
    out5 = _sc_lookup(embedding_weights, idx_t)
    return out5.transpose(2, 4, 0, 1, 3).reshape(B0, B1, D)


# speedup vs baseline: 1.3302x; 1.3302x over previous
import functools

import jax
import jax.numpy as jnp
from jax import lax
from jax.experimental import pallas as pl
from jax.experimental.pallas import tpu as pltpu
from jax.experimental.pallas import tpu_sc as plsc

_INFO = plsc.get_sparse_core_info()
_NC = _INFO.num_cores
_NS = _INFO.num_subcores
_NW = _NC * _NS

_C = 256


@jax.jit
def _sc_lookup(table, idx_t):
    S, T = idx_t.shape
    V, D = table.shape
    t_per_w = T // _NW
    n_chunks = S * (t_per_w // _C)
    mesh = plsc.VectorSubcoreMesh(core_axis_name="c", subcore_axis_name="s")

    @functools.partial(
        pl.kernel,
        mesh=mesh,
        compiler_params=pltpu.CompilerParams(
            use_tc_tiling_on_sc=False, needs_layout_passes=False),
        out_type=jax.ShapeDtypeStruct((S, D // 8, T // 128, 8, 128),
                                      jnp.float32),
        scratch_types=[
            pltpu.VMEM((2, 1, _C), jnp.int32),
            pltpu.VMEM((_C, D), jnp.float32),
            pltpu.VMEM((_C, D), jnp.float32),
            pltpu.VMEM((_C * (D + 1),), jnp.float32),
            pltpu.VMEM((D // 8, _C // 128, 8, 128), jnp.float32),
            pltpu.VMEM((D // 8, _C // 128, 8, 128), jnp.float32),
            pltpu.SemaphoreType.DMA,
            pltpu.SemaphoreType.DMA,
            pltpu.SemaphoreType.DMA,
            pltpu.SemaphoreType.DMA,
            pltpu.SemaphoreType.DMA,
            pltpu.SemaphoreType.DMA,
        ],
    )
    def k(table_hbm, idx_hbm, out_hbm, idx_v, rows0, rows1, rp, tb0, tb1,
          gs0, gs1, ws0, ws1, is0, is1):
        wid = lax.axis_index("s") * _NC + lax.axis_index("c")
        t0w = wid * t_per_w
        rows = (rows0, rows1)
        tbs = (tb0, tb1)
        gsem = (gs0, gs1)
        wsem = (ws0, ws1)
        isem = (is0, is1)
        cpb = t_per_w // _C
        lane = lax.iota(jnp.int32, 16)
        ones = jnp.ones((16,), jnp.int32)

        def idx_load(n, b):
            s = n // cpb
            toff = t0w + (n % cpb) * _C
            return pltpu.make_async_copy(
                idx_hbm.at[pl.ds(s, 1), pl.ds(toff, _C)], idx_v.at[b],
                isem[b])

        def gather(b):
            return pltpu.make_async_copy(
                table_hbm.at[idx_v.at[b, 0]], rows[b], gsem[b])

        def block_write(n, b):
            s = n // cpb
            tt0 = wid * (t_per_w // 128) + (n % cpb) * (_C // 128)
            return pltpu.make_async_copy(
                tbs[b], out_hbm.at[s, :, pl.ds(tt0, _C // 128), :, :],
                wsem[b])

        P = D + 1

        def transpose(b):
            rb = rows[b]
            tb = tbs[b]

            def repitch(j4, carry):
                for jj in range(4):
                    j = j4 * 4 + jj
                    for q in range(D // 16):
                        rp[pl.ds(j * P + q * 16, 16)] = rb[j, pl.ds(q * 16, 16)]
                return carry

            lax.fori_loop(0, _C // 4, repitch, 0)
            laneP = lane * P

            def body(m, carry):
                g = m >> 4
                tj = (m >> 3) & 1
                dd = m & 7
                sbase = laneP + (tj * 128 * P + 8 * g + dd)
                for kk in range(8):
                    v = plsc.load_gather(rp, [sbase + kk * (16 * P)])
                    tb[g, tj, dd, pl.ds(kk * 16, 16)] = v
                return carry

            lax.fori_loop(0, (D // 8) * (_C // 128) * 8, body, 0)

        idx_load(0, 0).start()
        idx_load(1, 1).start()
        idx_load(0, 0).wait()
        gather(0).start()

        def chunk(n, b):
            gather(b).wait()

            @pl.when(n + 2 < n_chunks)
            def _():
                idx_load(n + 2, b).start()

            @pl.when(n + 1 < n_chunks)
            def _():
                idx_load(n + 1, 1 - b).wait()
                gather(1 - b).start()

            @pl.when(n >= 2)
            def _():
                block_write(n - 2, b).wait()
            transpose(b)
            block_write(n, b).start()

        def pair(j, carry):
            chunk(2 * j, 0)
            chunk(2 * j + 1, 1)
            return carry

        lax.fori_loop(0, n_chunks // 2, pair, 0)
        block_write(n_chunks - 2, 0).wait()
        block_write(n_chunks - 1, 1).wait()

    return k(table, idx_t)


def kernel(inputs, embedding_weights):
    B0, B1 = inputs.shape
    V, D = embedding_weights.shape
    idx_t = jnp.swapaxes(inputs, 0, 1).astype(jnp.int32)
    out5 = _sc_lookup(embedding_weights, idx_t)
    return out5.transpose(2, 4, 0, 1, 3).reshape(B0, B1, D)


# speedup vs baseline: 1.3429x; 1.0095x over previous
import functools

import jax
import jax.numpy as jnp
from jax import lax
from jax.experimental import pallas as pl
from jax.experimental.pallas import tpu as pltpu
from jax.experimental.pallas import tpu_sc as plsc

_INFO = plsc.get_sparse_core_info()
_NC = _INFO.num_cores
_NS = _INFO.num_subcores
_NW = _NC * _NS

_C = 256


@jax.jit
def _sc_lookup(table, idx_t):
    S, T = idx_t.shape
    V, D = table.shape
    t_per_w = T // _NW
    n_chunks = S * (t_per_w // _C)
    mesh = plsc.VectorSubcoreMesh(core_axis_name="c", subcore_axis_name="s")

    @functools.partial(
        pl.kernel,
        mesh=mesh,
        compiler_params=pltpu.CompilerParams(
            use_tc_tiling_on_sc=False, needs_layout_passes=False),
        out_type=jax.ShapeDtypeStruct((S, D // 8, T // 128, 8, 128),
                                      jnp.float32),
        scratch_types=[
            pltpu.VMEM((2, 1, _C), jnp.int32),
            pltpu.VMEM((_C, D), jnp.float32),
            pltpu.VMEM((_C, D), jnp.float32),
            pltpu.VMEM((_C * (D + 1) + 64,), jnp.float32),
            pltpu.VMEM((D // 8, _C // 128, 8, 128), jnp.float32),
            pltpu.VMEM((D // 8, _C // 128, 8, 128), jnp.float32),
            pltpu.SemaphoreType.DMA,
            pltpu.SemaphoreType.DMA,
            pltpu.SemaphoreType.DMA,
            pltpu.SemaphoreType.DMA,
            pltpu.SemaphoreType.DMA,
            pltpu.SemaphoreType.DMA,
        ],
    )
    def k(table_hbm, idx_hbm, out_hbm, idx_v, rows0, rows1, rp, tb0, tb1,
          gs0, gs1, ws0, ws1, is0, is1):
        wid = lax.axis_index("s") * _NC + lax.axis_index("c")
        t0w = wid * t_per_w
        rows = (rows0, rows1)
        tbs = (tb0, tb1)
        gsem = (gs0, gs1)
        wsem = (ws0, ws1)
        isem = (is0, is1)
        cpb = t_per_w // _C
        lane = lax.iota(jnp.int32, 16)
        ones = jnp.ones((16,), jnp.int32)

        def idx_load(n, b):
            s = n // cpb
            toff = t0w + (n % cpb) * _C
            return pltpu.make_async_copy(
                idx_hbm.at[pl.ds(s, 1), pl.ds(toff, _C)], idx_v.at[b],
                isem[b])

        def gather(b):
            return pltpu.make_async_copy(
                table_hbm.at[idx_v.at[b, 0]], rows[b], gsem[b])

        def block_write(n, b):
            s = n // cpb
            tt0 = wid * (t_per_w // 128) + (n % cpb) * (_C // 128)
            return pltpu.make_async_copy(
                tbs[b], out_hbm.at[s, :, pl.ds(tt0, _C // 128), :, :],
                wsem[b])

        P = D + 1
        laneP_dd = [lane * P + dd for dd in range(8)]

        def transpose(b):
            rb = rows[b]
            tb = tbs[b]

            def repitch(j8, carry):
                j0 = j8 * 8
                for jj in range(8):
                    j = j0 + jj
                    for q in range(D // 16):
                        rp[pl.ds(j * P + q * 16, 16)] = rb[jj + j0,
                                                           pl.ds(q * 16, 16)]
                return carry

            lax.fori_loop(0, _C // 8, repitch, 0)

            def body(m, carry):
                tj = m >> 3
                kk = m & 7
                sofs = pl.multiple_of((tj * 128 + kk * 16) * P, 8)
                dbase = kk * 16
                for g in range(D // 8):
                    rs = rp.at[pl.ds(sofs + 8 * g, 1024)]
                    for dd in range(8):
                        v = plsc.load_gather(rs, [laneP_dd[dd]])
                        tb[g, tj, dd, pl.ds(dbase, 16)] = v
                return carry

            lax.fori_loop(0, (_C // 128) * 8, body, 0)

        idx_load(0, 0).start()
        idx_load(1, 1).start()
        idx_load(0, 0).wait()
        gather(0).start()

        def chunk(n, b):
            gather(b).wait()

            @pl.when(n + 2 < n_chunks)
            def _():
                idx_load(n + 2, b).start()

            @pl.when(n + 1 < n_chunks)
            def _():
                idx_load(n + 1, 1 - b).wait()
                gather(1 - b).start()

            @pl.when(n >= 2)
            def _():
                block_write(n - 2, b).wait()
            transpose(b)
            block_write(n, b).start()

        def pair(j, carry):
            chunk(2 * j, 0)
            chunk(2 * j + 1, 1)
            return carry

        lax.fori_loop(0, n_chunks // 2, pair, 0)
        block_write(n_chunks - 2, 0).wait()
        block_write(n_chunks - 1, 1).wait()

    return k(table, idx_t)


def kernel(inputs, embedding_weights):
    B0, B1 = inputs.shape
    V, D = embedding_weights.shape
    idx_t = jnp.swapaxes(inputs, 0, 1).astype(jnp.int32)
    out5 = _sc_lookup(embedding_weights, idx_t)
    return out5.transpose(2, 4, 0, 1, 3).reshape(B0, B1, D)


# speedup vs baseline: 2.1585x; 1.6074x over previous
import functools

import jax
import jax.numpy as jnp
from jax import lax
from jax.experimental import pallas as pl
from jax.experimental.pallas import tpu as pltpu
from jax.experimental.pallas import tpu_sc as plsc

_INFO = plsc.get_sparse_core_info()
_NC = _INFO.num_cores
_NS = _INFO.num_subcores
_NW = _NC * _NS

_C = 256


@jax.jit
def _sc_lookup(table, idx_t):
    S, T = idx_t.shape
    V, D = table.shape
    t_per_w = T // _NW
    n_chunks = S * (t_per_w // _C)
    mesh = plsc.VectorSubcoreMesh(core_axis_name="c", subcore_axis_name="s")

    @functools.partial(
        pl.kernel,
        mesh=mesh,
        compiler_params=pltpu.CompilerParams(
            use_tc_tiling_on_sc=False, needs_layout_passes=False),
        out_type=jax.ShapeDtypeStruct((S, D // 8, T // 128, 8, 128),
                                      jnp.float32),
        scratch_types=[
            pltpu.VMEM((2, 1, _C), jnp.int32),
            pltpu.VMEM((_C, D), jnp.float32),
            pltpu.VMEM((_C, D), jnp.float32),
            pltpu.VMEM((_C * (D + 1) + 64,), jnp.float32),
            pltpu.VMEM((D // 8, _C // 128, 8, 128), jnp.float32),
            pltpu.VMEM((D // 8, _C // 128, 8, 128), jnp.float32),
            pltpu.SemaphoreType.DMA,
            pltpu.SemaphoreType.DMA,
            pltpu.SemaphoreType.DMA,
            pltpu.SemaphoreType.DMA,
            pltpu.SemaphoreType.DMA,
            pltpu.SemaphoreType.DMA,
        ],
    )
    def k(table_hbm, idx_hbm, out_hbm, idx_v, rows0, rows1, rp, tb0, tb1,
          gs0, gs1, ws0, ws1, is0, is1):
        wid = lax.axis_index("s") * _NC + lax.axis_index("c")
        t0w = wid * t_per_w
        rows = (rows0, rows1)
        tbs = (tb0, tb1)
        gsem = (gs0, gs1)
        wsem = (ws0, ws1)
        isem = (is0, is1)
        cpb = t_per_w // _C
        lane = lax.iota(jnp.int32, 16)
        ones = jnp.ones((16,), jnp.int32)

        def idx_load(n, b):
            s = n // cpb
            toff = t0w + (n % cpb) * _C
            return pltpu.make_async_copy(
                idx_hbm.at[pl.ds(s, 1), pl.ds(toff, _C)], idx_v.at[b],
                isem[b])

        def gather(b):
            return pltpu.make_async_copy(
                table_hbm.at[idx_v.at[b, 0]], rows[b], gsem[b])

        def block_write(n, b):
            s = n // cpb
            tt0 = wid * (t_per_w // 128) + (n % cpb) * (_C // 128)
            return pltpu.make_async_copy(
                tbs[b], out_hbm.at[s, :, pl.ds(tt0, _C // 128), :, :],
                wsem[b])

        P = D + 1
        laneP_dd = [lane * P + dd for dd in range(8)]

        def transpose(b):
            rb = rows[b]
            tb = tbs[b]

            def repitch(j8, carry):
                j0 = j8 * 8
                vs = [rb[jj + j0, pl.ds(q * 16, 16)]
                      for jj in range(8) for q in range(D // 16)]
                i = 0
                for jj in range(8):
                    for q in range(D // 16):
                        rp[pl.ds((j0 + jj) * P + q * 16, 16)] = vs[i]
                        i += 1
                return carry

            lax.fori_loop(0, _C // 8, repitch, 0)

            def body(m, carry):
                tj = m >> 3
                kk = m & 7
                sofs = pl.multiple_of((tj * 128 + kk * 16) * P, 8)
                dbase = kk * 16
                for g2 in range(D // 16):
                    rs0 = rp.at[pl.ds(sofs + 16 * g2, 1024)]
                    rs1 = rp.at[pl.ds(sofs + 16 * g2 + 8, 1024)]
                    vs = ([plsc.load_gather(rs0, [laneP_dd[dd]])
                           for dd in range(8)] +
                          [plsc.load_gather(rs1, [laneP_dd[dd]])
                           for dd in range(8)])
                    for h in range(2):
                        for dd in range(8):
                            tb[2 * g2 + h, tj, dd, pl.ds(dbase, 16)] = \
                                vs[8 * h + dd]
                return carry

            lax.fori_loop(0, (_C // 128) * 8, body, 0)

        idx_load(0, 0).start()
        idx_load(1, 1).start()
        idx_load(0, 0).wait()
        gather(0).start()

        def chunk(n, b):
            gather(b).wait()

            @pl.when(n + 2 < n_chunks)
            def _():
                idx_load(n + 2, b).start()

            @pl.when(n + 1 < n_chunks)
            def _():
                idx_load(n + 1, 1 - b).wait()
                gather(1 - b).start()

            @pl.when(n >= 2)
            def _():
                block_write(n - 2, b).wait()
            transpose(b)
            block_write(n, b).start()

        def pair(j, carry):
            chunk(2 * j, 0)
            chunk(2 * j + 1, 1)
            return carry

        lax.fori_loop(0, n_chunks // 2, pair, 0)
        block_write(n_chunks - 2, 0).wait()
        block_write(n_chunks - 1, 1).wait()

    return k(table, idx_t)


def kernel(inputs, embedding_weights):
    B0, B1 = inputs.shape
    V, D = embedding_weights.shape
    idx_t = jnp.swapaxes(inputs, 0, 1).astype(jnp.int32)
    out5 = _sc_lookup(embedding_weights, idx_t)
    return out5.transpose(2, 4, 0, 1, 3).reshape(B0, B1, D)


# speedup vs baseline: 2.3355x; 1.0820x over previous
import functools

import jax
import jax.numpy as jnp
from jax import lax
from jax.experimental import pallas as pl
from jax.experimental.pallas import tpu as pltpu
from jax.experimental.pallas import tpu_sc as plsc

_INFO = plsc.get_sparse_core_info()
_NC = _INFO.num_cores
_NS = _INFO.num_subcores
_NW = _NC * _NS

_C = 256


@jax.jit
def _sc_lookup(table, idx_t):
    S, T = idx_t.shape
    V, D = table.shape
    t_per_w = T // _NW
    n_chunks = S * (t_per_w // _C)
    mesh = plsc.VectorSubcoreMesh(core_axis_name="c", subcore_axis_name="s")

    @functools.partial(
        pl.kernel,
        mesh=mesh,
        compiler_params=pltpu.CompilerParams(
            use_tc_tiling_on_sc=False, needs_layout_passes=False),
        out_type=jax.ShapeDtypeStruct((S, D // 8, T // 128, 8, 128),
                                      jnp.float32),
        scratch_types=[
            pltpu.VMEM((2, 1, _C), jnp.int32),
            pltpu.VMEM((_C, D), jnp.float32),
            pltpu.VMEM((_C, D), jnp.float32),
            pltpu.VMEM((_C * (D + 1) + 64,), jnp.float32),
            pltpu.VMEM((D // 8, _C // 128, 8, 128), jnp.float32),
            pltpu.VMEM((D // 8, _C // 128, 8, 128), jnp.float32),
            pltpu.SemaphoreType.DMA,
            pltpu.SemaphoreType.DMA,
            pltpu.SemaphoreType.DMA,
            pltpu.SemaphoreType.DMA,
            pltpu.SemaphoreType.DMA,
            pltpu.SemaphoreType.DMA,
        ],
    )
    def k(table_hbm, idx_hbm, out_hbm, idx_v, rows0, rows1, rp, tb0, tb1,
          gs0, gs1, ws0, ws1, is0, is1):
        wid = lax.axis_index("s") * _NC + lax.axis_index("c")
        t0w = wid * t_per_w
        rows = (rows0, rows1)
        tbs = (tb0, tb1)
        gsem = (gs0, gs1)
        wsem = (ws0, ws1)
        isem = (is0, is1)
        cpb = t_per_w // _C
        lane = lax.iota(jnp.int32, 16)
        ones = jnp.ones((16,), jnp.int32)

        def idx_load(n, b):
            s = n // cpb
            toff = t0w + (n % cpb) * _C
            return pltpu.make_async_copy(
                idx_hbm.at[pl.ds(s, 1), pl.ds(toff, _C)], idx_v.at[b],
                isem[b])

        def gather(b):
            return pltpu.make_async_copy(
                table_hbm.at[idx_v.at[b, 0]], rows[b], gsem[b])

        def block_write(n, b):
            s = n // cpb
            tt0 = wid * (t_per_w // 128) + (n % cpb) * (_C // 128)
            return pltpu.make_async_copy(
                tbs[b], out_hbm.at[s, :, pl.ds(tt0, _C // 128), :, :],
                wsem[b])

        P = D + 1
        laneP_dd = [lane * P + dd for dd in range(8)]

        def transpose(b):
            rb = rows[b]
            tb = tbs[b]

            @plsc.parallel_loop(0, _C // 8)
            def repitch(j8):
                j0 = j8 * 8
                vs = [rb[jj + j0, pl.ds(q * 16, 16)]
                      for jj in range(8) for q in range(D // 16)]
                i = 0
                for jj in range(8):
                    for q in range(D // 16):
                        rp[pl.ds((j0 + jj) * P + q * 16, 16)] = vs[i]
                        i += 1

            @plsc.parallel_loop(0, (_C // 128) * 8)
            def body(m):
                tj = m >> 3
                kk = m & 7
                sofs = pl.multiple_of((tj * 128 + kk * 16) * P, 8)
                dbase = kk * 16
                for g2 in range(D // 16):
                    rs0 = rp.at[pl.ds(sofs + 16 * g2, 1024)]
                    rs1 = rp.at[pl.ds(sofs + 16 * g2 + 8, 1024)]
                    vs = ([plsc.load_gather(rs0, [laneP_dd[dd]])
                           for dd in range(8)] +
                          [plsc.load_gather(rs1, [laneP_dd[dd]])
                           for dd in range(8)])
                    for h in range(2):
                        for dd in range(8):
                            tb[2 * g2 + h, tj, dd, pl.ds(dbase, 16)] = \
                                vs[8 * h + dd]

        idx_load(0, 0).start()
        idx_load(1, 1).start()
        idx_load(0, 0).wait()
        gather(0).start()

        def chunk(n, b):
            gather(b).wait()

            @pl.when(n + 2 < n_chunks)
            def _():
                idx_load(n + 2, b).start()

            @pl.when(n + 1 < n_chunks)
            def _():
                idx_load(n + 1, 1 - b).wait()
                gather(1 - b).start()

            @pl.when(n >= 2)
            def _():
                block_write(n - 2, b).wait()
            transpose(b)
            block_write(n, b).start()

        def pair(j, carry):
            chunk(2 * j, 0)
            chunk(2 * j + 1, 1)
            return carry

        lax.fori_loop(0, n_chunks // 2, pair, 0)
        block_write(n_chunks - 2, 0).wait()
        block_write(n_chunks - 1, 1).wait()

    return k(table, idx_t)


def kernel(inputs, embedding_weights):
    B0, B1 = inputs.shape
    V, D = embedding_weights.shape
    idx_t = jnp.swapaxes(inputs, 0, 1).astype(jnp.int32)
    out5 = _sc_lookup(embedding_weights, idx_t)
    return out5.transpose(2, 4, 0, 1, 3).reshape(B0, B1, D)
